# Initial kernel scaffold; baseline (speedup 1.0000x reference)
import jax
import jax.numpy as jnp
from jax.experimental import pallas as pl


def kernel(keys, points, feats, W1, b1, W2, b2, W3, b3, F1, fb1, F2, fb2):
    raise NotImplementedError("write your pallas kernel here")


# speedup vs baseline: 1.4539x; 1.4539x over previous
import functools

import jax
import jax.numpy as jnp
from jax import lax
from jax.experimental import pallas as pl
from jax.experimental.pallas import tpu as pltpu

B, K, N, DIM = 4, 1024, 2048, 3
C_IN, C_MID, NB, C_OUT = 64, 16, 32, 128
BK = B * K
QB = 128
NBLK = BK // QB


def _tc_body(rel4_ref, nf_ref, W1p_ref, b1t_ref, W2p_ref, b2t_ref,
             W3p_ref, b3t_ref, F1r_ref, fb1_ref, F2_ref, fb2_ref, out_ref,
             e_ref):
    rel4 = rel4_ref[...]
    h = jnp.maximum(jnp.dot(rel4, W1p_ref[...],
                            preferred_element_type=jnp.float32)
                    + b1t_ref[...][None, :], 0.0)
    h = jnp.maximum(jnp.dot(h, W2p_ref[...],
                            preferred_element_type=jnp.float32)
                    + b2t_ref[...][None, :], 0.0)
    m = jnp.dot(h, W3p_ref[...],
                preferred_element_type=jnp.float32) + b3t_ref[...][None, :]
    nf = nf_ref[...]
    e_ref[...] = jnp.zeros((QB, C_MID, C_IN), jnp.float32)
    for j in range(NB):
        mj = m[:, j * C_MID:(j + 1) * C_MID]
        fj = nf[:, j * C_IN:(j + 1) * C_IN]
        e_ref[...] += mj[:, :, None] * fj[:, None, :]
    g = fb1_ref[...][None, :]
    for a in range(C_MID):
        g = g + jnp.dot(e_ref[:, a, :], F1r_ref[a],
                        preferred_element_type=jnp.float32)
    g = jnp.maximum(g, 0.0)
    out_ref[...] = jnp.dot(g, F2_ref[...],
                           preferred_element_type=jnp.float32) \
        + fb2_ref[...][None, :]


@jax.jit
def _tc_mlp(rel4, nf2, W1p, b1t, W2p, b2t, W3p, b3t, F1r, fb1, F2, fb2):
    fixed = lambda *shape: pl.BlockSpec(shape, lambda i: (0,) * len(shape))
    return pl.pallas_call(
        _tc_body,
        grid=(NBLK,),
        in_specs=[
            pl.BlockSpec((QB, 128), lambda i: (i, 0)),
            pl.BlockSpec((QB, NB * C_IN), lambda i: (i, 0)),
            fixed(128, 1024), fixed(1024), fixed(1024, 1024), fixed(1024),
            fixed(1024, 512), fixed(512), fixed(C_MID, C_IN, 256), fixed(256),
            fixed(256, C_OUT), fixed(C_OUT),
        ],
        out_specs=pl.BlockSpec((QB, C_OUT), lambda i: (i, 0)),
        out_shape=jax.ShapeDtypeStruct((BK, C_OUT), jnp.float32),
        scratch_shapes=[pltpu.VMEM((QB, C_MID, C_IN), jnp.float32)],
    )(rel4, nf2, W1p, b1t, W2p, b2t, W3p, b3t, F1r, fb1, F2, fb2)


def _prep_weights(W1, b1, W2, b2, W3, b3, F1):
    W1pad = jnp.concatenate([W1, jnp.zeros((1, 32), jnp.float32)], axis=0)
    W1p = jnp.kron(jnp.eye(NB, dtype=jnp.float32), W1pad)
    W2p = jnp.kron(jnp.eye(NB, dtype=jnp.float32), W2)
    W3p = jnp.kron(jnp.eye(NB, dtype=jnp.float32), W3)
    b1t = jnp.tile(b1, NB)
    b2t = jnp.tile(b2, NB)
    b3t = jnp.tile(b3, NB)
    F1r = F1.reshape(C_MID, C_IN, 256)
    return W1p, b1t, W2p, b2t, W3p, b3t, F1r


def _topk_gather_jax(keys, points, feats):
    dist_vec = points[:, None, :, :] - keys[:, :, None, :]
    dist = jnp.sum(dist_vec * dist_vec, axis=-1)
    _, idxs = lax.top_k(-dist, NB)
    nrel = jnp.take_along_axis(dist_vec, idxs[:, :, :, None], axis=2)
    nf = jnp.take_along_axis(feats[:, None, :, :], idxs[:, :, :, None], axis=2)
    rel4 = jnp.concatenate(
        [nrel, jnp.zeros(nrel.shape[:3] + (1,), jnp.float32)], axis=-1)
    rel4 = rel4.reshape(BK, NB * 4)
    nf2 = nf.reshape(BK, NB * C_IN)
    return rel4, nf2


def kernel(keys, points, feats, W1, b1, W2, b2, W3, b3, F1, fb1, F2, fb2):
    rel4, nf2 = _topk_gather_jax(keys, points, feats)
    W1p, b1t, W2p, b2t, W3p, b3t, F1r = _prep_weights(W1, b1, W2, b2, W3, b3, F1)
    out = _tc_mlp(rel4, nf2, W1p, b1t, W2p, b2t, W3p, b3t, F1r, fb1, F2, fb2)
    return out.reshape(B, K, C_OUT)


# speedup vs baseline: 3.7793x; 2.5993x over previous
import functools

import jax
import jax.numpy as jnp
from jax import lax
from jax.experimental import pallas as pl
from jax.experimental.pallas import tpu as pltpu
from jax.experimental.pallas import tpu_sc as plsc

B, K, N, DIM = 4, 1024, 2048, 3
C_IN, C_MID, NB, C_OUT = 64, 16, 32, 128
BK = B * K

NW = 32
QPT = BK // NW
GRP = 8
NGRP = QPT // GRP
NV = N // 16
FP = 128

QB = 128
NBLK = BK // QB


def _sc_body(keys_hbm, pts_hbm, feats_hbm, nf_out, rel_out,
             pts_v, kq_v, bits8, hist8, chist8, idx8, gidx8, relb, fbuf,
             semg):
    iota = lax.iota(jnp.int32, 16)
    ones = jnp.ones((16,), jnp.int32)
    fifteen = jnp.full((16,), 15, jnp.int32)
    t32 = jnp.full((16,), NB, jnp.int32)
    zf16 = jnp.zeros((16,), jnp.float32)
    zi16 = jnp.zeros((16,), jnp.int32)

    wid = lax.axis_index("s") * 2 + lax.axis_index("c")
    b = wid // (K // QPT)
    kbase = (wid % (K // QPT)) * QPT
    feat_off = b * N

    pltpu.sync_copy(pts_hbm.at[b], pts_v)
    pltpu.sync_copy(keys_hbm.at[b, :, pl.ds(kbase, QPT)], kq_v)

    gconsts = [jnp.full((16,), g, jnp.int32) for g in range(GRP)]

    def scan_radix(g, thresh):
        cvec = chist8[g, pl.ds(0, 16)]
        cumc = plsc.cumsum(cvec)
        cstar = plsc.all_reduce_population_count(cumc < thresh)
        belowc = jnp.take(cumc - cvec, cstar)
        hv = plsc.load_gather(hist8, [gconsts[g], cstar * 16 + iota])
        cumf = plsc.cumsum(hv) + belowc
        bloc = plsc.all_reduce_population_count(cumf < thresh)
        below = jnp.take(cumf - hv, bloc)
        return cstar * 16 + bloc, below

    def zero_hist(g):
        for t in range(16):
            hist8[g, pl.ds(t * 16, 16)] = zi16
        chist8[g, pl.ds(0, 16)] = zi16

    def refine_pass(g, pshift, prefix, bshift):
        def body(i, c):
            bt = bits8[g, pl.ds(i * 16, 16)]
            match = lax.shift_right_logical(bt, pshift) == prefix
            binp = jnp.bitwise_and(lax.shift_right_logical(bt, bshift), 255)
            plsc.addupdate_scatter(hist8, [gconsts[g], binp], ones,
                                   mask=match)
            plsc.addupdate_scatter(
                chist8, [gconsts[g], lax.shift_right_logical(binp, 4)],
                ones, mask=match)
            return c
        lax.fori_loop(0, NV, body, 0, unroll=2)

    def grp_body(grp, carry):
        q0 = grp * GRP
        qs = []
        for g in range(GRP):
            qloc = jnp.full((16,), q0 + g, jnp.int32)
            qs.append([plsc.load_gather(kq_v, [jnp.full((16,), d, jnp.int32),
                                               qloc]) for d in range(3)])
        for g in range(GRP):
            zero_hist(g)

        def p1_body(i, c):
            sl = pl.ds(i * 16, 16)
            px = pts_v[0, sl]
            py = pts_v[1, sl]
            pz = pts_v[2, sl]
            for g in range(GRP):
                dx = px - qs[g][0]
                dy = py - qs[g][1]
                dz = pz - qs[g][2]
                d2 = dx * dx + dy * dy + dz * dz
                bt = plsc.bitcast(d2, jnp.int32)
                bits8[g, sl] = bt
                plsc.addupdate_scatter(
                    hist8, [gconsts[g], lax.shift_right_logical(bt, 24)],
                    ones)
                plsc.addupdate_scatter(
                    chist8, [gconsts[g], lax.shift_right_logical(bt, 28)],
                    ones)
            return c
        lax.fori_loop(0, NV, p1_body, 0)

        for g in range(GRP):
            b0, c1 = scan_radix(g, t32)
            t2 = t32 - c1
            zero_hist(g)
            refine_pass(g, 24, b0, 16)
            b1, c2 = scan_radix(g, t2)
            t3 = t2 - c2
            zero_hist(g)
            refine_pass(g, 16, b0 * 256 + b1, 8)
            b2, c3 = scan_radix(g, t3)
            t4 = t3 - c3
            zero_hist(g)
            refine_pass(g, 8, (b0 * 256 + b1) * 256 + b2, 0)
            b3, _ = scan_radix(g, t4)
            T = ((b0 * 256 + b1) * 256 + b2) * 256 + b3

            def lt_body(i, cur):
                bt = bits8[g, pl.ds(i * 16, 16)]
                m = bt < T
                cs = plsc.cumsum(m.astype(jnp.int32))
                pos = cur + cs - 1
                plsc.store_scatter(idx8, [gconsts[g], pos], i * 16 + iota,
                                   mask=m)
                return cur + jnp.take(cs, fifteen)
            cur = lax.fori_loop(0, NV, lt_body, zi16, unroll=2)

            def eq_body(i, cur):
                bt = bits8[g, pl.ds(i * 16, 16)]
                m = bt == T
                cs = plsc.cumsum(m.astype(jnp.int32))
                pos = cur + cs - 1
                keep = jnp.logical_and(m, pos < t32)
                plsc.store_scatter(idx8, [gconsts[g], pos], i * 16 + iota,
                                   mask=keep)
                return cur + jnp.take(cs, fifteen)
            lax.fori_loop(0, NV, eq_body, cur, unroll=2)

            i0 = idx8[g, pl.ds(0, 16)]
            i1 = idx8[g, pl.ds(16, 16)]
            for d in range(3):
                dsp = jnp.full((16,), d, jnp.int32)
                v0 = plsc.load_gather(pts_v, [dsp, i0]) - qs[g][d]
                v1 = plsc.load_gather(pts_v, [dsp, i1]) - qs[g][d]
                plsc.store_scatter(relb, [gconsts[g], iota * 4 + d], v0)
                plsc.store_scatter(relb, [gconsts[g], (iota + 16) * 4 + d],
                                   v1)
            plsc.store_scatter(relb, [gconsts[g], iota * 4 + 3], zf16)
            plsc.store_scatter(relb, [gconsts[g], (iota + 16) * 4 + 3], zf16)
            gidx8[g, pl.ds(0, 16)] = i0 + feat_off
            gidx8[g, pl.ds(16, 16)] = i1 + feat_off

        descs = [pltpu.async_copy(feats_hbm.at[gidx8.at[g]], fbuf.at[g],
                                  semg) for g in range(GRP)]
        for dsc in descs:
            dsc.wait()
        row0 = wid * QPT + q0
        pltpu.sync_copy(fbuf, nf_out.at[pl.ds(row0, GRP)])
        pltpu.sync_copy(relb, rel_out.at[pl.ds(row0, GRP)])
        return carry

    lax.fori_loop(0, NGRP, grp_body, 0)


def _sc_select_gather(keys_t, points_t, feats_pad):
    mesh = plsc.VectorSubcoreMesh(core_axis_name="c", subcore_axis_name="s",
                                  num_cores=2, num_subcores=16)
    return pl.kernel(
        _sc_body,
        out_type=(jax.ShapeDtypeStruct((BK, NB, FP), jnp.float32),
                  jax.ShapeDtypeStruct((BK, 128), jnp.float32)),
        mesh=mesh,
        compiler_params=pltpu.CompilerParams(needs_layout_passes=False),
        scratch_types=[
            pltpu.VMEM((3, N), jnp.float32),
            pltpu.VMEM((3, QPT), jnp.float32),
            pltpu.VMEM((GRP, N), jnp.int32),
            pltpu.VMEM((GRP, 256), jnp.int32),
            pltpu.VMEM((GRP, 16), jnp.int32),
            pltpu.VMEM((GRP, 64), jnp.int32),
            pltpu.VMEM((GRP, NB), jnp.int32),
            pltpu.VMEM((GRP, 128), jnp.float32),
            pltpu.VMEM((GRP, NB, FP), jnp.float32),
            pltpu.SemaphoreType.DMA,
        ],
    )(keys_t, points_t, feats_pad)


def _tc_body(rel4_ref, nf_ref, W1p_ref, b1t_ref, W2p_ref, b2t_ref,
             W3p_ref, b3t_ref, F1r_ref, fb1_ref, F2_ref, fb2_ref, out_ref,
             e_ref):
    rel4 = rel4_ref[...]
    h = jnp.maximum(jnp.dot(rel4, W1p_ref[...],
                            preferred_element_type=jnp.float32)
                    + b1t_ref[...][None, :], 0.0)
    h = jnp.maximum(jnp.dot(h, W2p_ref[...],
                            preferred_element_type=jnp.float32)
                    + b2t_ref[...][None, :], 0.0)
    m = jnp.dot(h, W3p_ref[...],
                preferred_element_type=jnp.float32) + b3t_ref[...][None, :]
    e_ref[...] = jnp.zeros((QB, C_MID, C_IN), jnp.float32)
    for j in range(NB):
        mj = m[:, j * C_MID:(j + 1) * C_MID]
        fj = nf_ref[:, j, 0:C_IN]
        e_ref[...] += mj[:, :, None] * fj[:, None, :]
    g = fb1_ref[...][None, :]
    for a in range(C_MID):
        g = g + jnp.dot(e_ref[:, a, :], F1r_ref[a],
                        preferred_element_type=jnp.float32)
    g = jnp.maximum(g, 0.0)
    out_ref[...] = jnp.dot(g, F2_ref[...],
                           preferred_element_type=jnp.float32) \
        + fb2_ref[...][None, :]


def _tc_mlp(rel4, nf3, W1p, b1t, W2p, b2t, W3p, b3t, F1r, fb1, F2, fb2):
    fixed = lambda *shape: pl.BlockSpec(shape, lambda i: (0,) * len(shape))
    return pl.pallas_call(
        _tc_body,
        grid=(NBLK,),
        in_specs=[
            pl.BlockSpec((QB, 128), lambda i: (i, 0)),
            pl.BlockSpec((QB, NB, FP), lambda i: (i, 0, 0)),
            fixed(128, 1024), fixed(1024), fixed(1024, 1024), fixed(1024),
            fixed(1024, 512), fixed(512), fixed(C_MID, C_IN, 256), fixed(256),
            fixed(256, C_OUT), fixed(C_OUT),
        ],
        out_specs=pl.BlockSpec((QB, C_OUT), lambda i: (i, 0)),
        out_shape=jax.ShapeDtypeStruct((BK, C_OUT), jnp.float32),
        scratch_shapes=[pltpu.VMEM((QB, C_MID, C_IN), jnp.float32)],
    )(rel4, nf3, W1p, b1t, W2p, b2t, W3p, b3t, F1r, fb1, F2, fb2)


def _prep_weights(W1, b1, W2, b2, W3, b3, F1):
    W1pad = jnp.concatenate([W1, jnp.zeros((1, 32), jnp.float32)], axis=0)
    W1p = jnp.kron(jnp.eye(NB, dtype=jnp.float32), W1pad)
    W2p = jnp.kron(jnp.eye(NB, dtype=jnp.float32), W2)
    W3p = jnp.kron(jnp.eye(NB, dtype=jnp.float32), W3)
    b1t = jnp.tile(b1, NB)
    b2t = jnp.tile(b2, NB)
    b3t = jnp.tile(b3, NB)
    F1r = F1.reshape(C_MID, C_IN, 256)
    return W1p, b1t, W2p, b2t, W3p, b3t, F1r


def kernel(keys, points, feats, W1, b1, W2, b2, W3, b3, F1, fb1, F2, fb2):
    keys_t = keys.transpose(0, 2, 1)
    points_t = points.transpose(0, 2, 1)
    feats_pad = jnp.pad(feats.reshape(B * N, C_IN),
                        ((0, 0), (0, FP - C_IN)))
    nf3, rel4 = _sc_select_gather(keys_t, points_t, feats_pad)
    W1p, b1t, W2p, b2t, W3p, b3t, F1r = _prep_weights(W1, b1, W2, b2, W3, b3,
                                                      F1)
    out = _tc_mlp(rel4, nf3, W1p, b1t, W2p, b2t, W3p, b3t, F1r, fb1, F2, fb2)
    return out.reshape(B, K, C_OUT)


# speedup vs baseline: 6.0936x; 1.6124x over previous
import functools

import jax
import jax.numpy as jnp
from jax import lax
from jax.experimental import pallas as pl
from jax.experimental.pallas import tpu as pltpu
from jax.experimental.pallas import tpu_sc as plsc

B, K, N, DIM = 4, 1024, 2048, 3
C_IN, C_MID, NB, C_OUT = 64, 16, 32, 128
BK = B * K

NW = 32
QPT = BK // NW
GRP = 8
NGRP = QPT // GRP
NV = N // 16
FP = 128

QB = 128
NBLK = BK // QB


def _sc_body(keys_hbm, pts_hbm, feats_hbm, nf_out, rel_out,
             pts_v, kq_v, bits8, hist8, chist8, idx8, gidx8, relb, fbuf,
             cb8, ci8, semg):
    iota = lax.iota(jnp.int32, 16)
    ones = jnp.ones((16,), jnp.int32)
    t32 = jnp.full((16,), NB, jnp.int32)
    zf16 = jnp.zeros((16,), jnp.float32)
    zi16 = jnp.zeros((16,), jnp.int32)

    wid = lax.axis_index("s") * 2 + lax.axis_index("c")
    b = wid // (K // QPT)
    kbase = (wid % (K // QPT)) * QPT
    feat_off = b * N

    pltpu.sync_copy(pts_hbm.at[b], pts_v)
    pltpu.sync_copy(keys_hbm.at[b, :, pl.ds(kbase, QPT)], kq_v)

    gconsts = [jnp.full((16,), g, jnp.int32) for g in range(GRP)]

    def scan_radix(g, thresh):
        cvec = chist8[g, pl.ds(0, 16)]
        cumc = plsc.cumsum(cvec)
        cstar = plsc.all_reduce_population_count(cumc < thresh)
        belowc = jnp.take(cumc - cvec, cstar)
        hv = plsc.load_gather(hist8, [gconsts[g], cstar * 16 + iota])
        cumf = plsc.cumsum(hv) + belowc
        bloc = plsc.all_reduce_population_count(cumf < thresh)
        below = jnp.take(cumf - hv, bloc)
        return cstar * 16 + bloc, below

    def zero_hist(g):
        for t in range(16):
            hist8[g, pl.ds(t * 16, 16)] = zi16
        chist8[g, pl.ds(0, 16)] = zi16

    def grp_body(grp, carry):
        q0 = grp * GRP
        qs = []
        for g in range(GRP):
            qloc = jnp.full((16,), q0 + g, jnp.int32)
            qs.append([plsc.load_gather(kq_v, [jnp.full((16,), d, jnp.int32),
                                               qloc]) for d in range(3)])
        for g in range(GRP):
            zero_hist(g)

        def p1_body(i, c):
            sl = pl.ds(i * 16, 16)
            px = pts_v[0, sl]
            py = pts_v[1, sl]
            pz = pts_v[2, sl]
            for g in range(GRP):
                dx = px - qs[g][0]
                dy = py - qs[g][1]
                dz = pz - qs[g][2]
                d2 = dx * dx + dy * dy + dz * dz
                bt = plsc.bitcast(d2, jnp.int32)
                bits8[g, sl] = bt
                plsc.addupdate_scatter(
                    hist8, [gconsts[g], lax.shift_right_logical(bt, 24)],
                    ones)
                plsc.addupdate_scatter(
                    chist8, [gconsts[g], lax.shift_right_logical(bt, 28)],
                    ones)
            return c
        lax.fori_loop(0, NV, p1_body, 0)

        for g in range(GRP):
            b0, c1 = scan_radix(g, t32)
            t2 = t32 - c1
            zero_hist(g)

            def p2_body(i, carry):
                cur_lt, cur_c = carry
                bt = bits8[g, pl.ds(i * 16, 16)]
                top8 = lax.shift_right_logical(bt, 24)
                match = top8 == b0
                lt8 = top8 < b0
                bin2 = jnp.bitwise_and(lax.shift_right_logical(bt, 16), 255)
                plsc.addupdate_scatter(hist8, [gconsts[g], bin2], ones,
                                       mask=match)
                plsc.addupdate_scatter(
                    chist8, [gconsts[g], lax.shift_right_logical(bin2, 4)],
                    ones, mask=match)
                lanes = i * 16 + iota
                cs_lt = plsc.cumsum(lt8.astype(jnp.int32))
                plsc.store_scatter(idx8, [gconsts[g], cur_lt + cs_lt - 1],
                                   lanes, mask=lt8)
                cs_c = plsc.cumsum(match.astype(jnp.int32))
                posc = cur_c + cs_c - 1
                plsc.store_scatter(cb8, [gconsts[g], posc], bt, mask=match)
                plsc.store_scatter(ci8, [gconsts[g], posc], lanes,
                                   mask=match)
                return (cur_lt + plsc.all_reduce_population_count(lt8),
                        cur_c + plsc.all_reduce_population_count(match))
            cur_lt, cur_c = lax.fori_loop(0, NV, p2_body, (zi16, zi16),
                                          unroll=2)
            m_sc = cur_c[0]
            nvc = (m_sc + 15) // 16

            b1, c2 = scan_radix(g, t2)
            t3 = t2 - c2
            zero_hist(g)

            def p3_body(i, c):
                bt = cb8[g, pl.ds(i * 16, 16)]
                inb = (i * 16 + iota) < m_sc
                nxt8 = jnp.bitwise_and(lax.shift_right_logical(bt, 16), 255)
                match = jnp.logical_and(nxt8 == b1, inb)
                bin3 = jnp.bitwise_and(lax.shift_right_logical(bt, 8), 255)
                plsc.addupdate_scatter(hist8, [gconsts[g], bin3], ones,
                                       mask=match)
                plsc.addupdate_scatter(
                    chist8, [gconsts[g], lax.shift_right_logical(bin3, 4)],
                    ones, mask=match)
                return c
            lax.fori_loop(0, nvc, p3_body, 0)
            b2, c3 = scan_radix(g, t3)
            t4 = t3 - c3
            zero_hist(g)

            def p4_body(i, c):
                bt = cb8[g, pl.ds(i * 16, 16)]
                inb = (i * 16 + iota) < m_sc
                nxt8 = jnp.bitwise_and(lax.shift_right_logical(bt, 16), 255)
                mid8 = jnp.bitwise_and(lax.shift_right_logical(bt, 8), 255)
                match = jnp.logical_and(
                    jnp.logical_and(nxt8 == b1, mid8 == b2), inb)
                bin4 = jnp.bitwise_and(bt, 255)
                plsc.addupdate_scatter(hist8, [gconsts[g], bin4], ones,
                                       mask=match)
                plsc.addupdate_scatter(
                    chist8, [gconsts[g], lax.shift_right_logical(bin4, 4)],
                    ones, mask=match)
                return c
            lax.fori_loop(0, nvc, p4_body, 0)
            b3, _ = scan_radix(g, t4)
            T = ((b0 * 256 + b1) * 256 + b2) * 256 + b3

            def lt_body(i, cur):
                bt = cb8[g, pl.ds(i * 16, 16)]
                ci = ci8[g, pl.ds(i * 16, 16)]
                inb = (i * 16 + iota) < m_sc
                m = jnp.logical_and(bt < T, inb)
                cs = plsc.cumsum(m.astype(jnp.int32))
                pos = cur + cs - 1
                plsc.store_scatter(idx8, [gconsts[g], pos], ci, mask=m)
                return cur + plsc.all_reduce_population_count(m)
            cur = lax.fori_loop(0, nvc, lt_body, cur_lt)

            def eq_body(i, cur):
                bt = cb8[g, pl.ds(i * 16, 16)]
                ci = ci8[g, pl.ds(i * 16, 16)]
                inb = (i * 16 + iota) < m_sc
                m = jnp.logical_and(bt == T, inb)
                cs = plsc.cumsum(m.astype(jnp.int32))
                pos = cur + cs - 1
                keep = jnp.logical_and(m, pos < t32)
                plsc.store_scatter(idx8, [gconsts[g], pos], ci, mask=keep)
                return cur + plsc.all_reduce_population_count(m)
            lax.fori_loop(0, nvc, eq_body, cur)

            i0 = idx8[g, pl.ds(0, 16)]
            i1 = idx8[g, pl.ds(16, 16)]
            for d in range(3):
                dsp = jnp.full((16,), d, jnp.int32)
                v0 = plsc.load_gather(pts_v, [dsp, i0]) - qs[g][d]
                v1 = plsc.load_gather(pts_v, [dsp, i1]) - qs[g][d]
                plsc.store_scatter(relb, [gconsts[g], iota * 4 + d], v0)
                plsc.store_scatter(relb, [gconsts[g], (iota + 16) * 4 + d],
                                   v1)
            plsc.store_scatter(relb, [gconsts[g], iota * 4 + 3], zf16)
            plsc.store_scatter(relb, [gconsts[g], (iota + 16) * 4 + 3], zf16)
            gidx8[g, pl.ds(0, 16)] = i0 + feat_off
            gidx8[g, pl.ds(16, 16)] = i1 + feat_off

        descs = [pltpu.async_copy(feats_hbm.at[gidx8.at[g]], fbuf.at[g],
                                  semg) for g in range(GRP)]
        for dsc in descs:
            dsc.wait()
        row0 = wid * QPT + q0
        pltpu.sync_copy(fbuf, nf_out.at[pl.ds(row0, GRP)])
        pltpu.sync_copy(relb, rel_out.at[pl.ds(row0, GRP)])
        return carry

    lax.fori_loop(0, NGRP, grp_body, 0)


def _sc_select_gather(keys_t, points_t, feats_pad):
    mesh = plsc.VectorSubcoreMesh(core_axis_name="c", subcore_axis_name="s",
                                  num_cores=2, num_subcores=16)
    return pl.kernel(
        _sc_body,
        out_type=(jax.ShapeDtypeStruct((BK, NB, FP), jnp.float32),
                  jax.ShapeDtypeStruct((BK, 128), jnp.float32)),
        mesh=mesh,
        compiler_params=pltpu.CompilerParams(needs_layout_passes=False),
        scratch_types=[
            pltpu.VMEM((3, N), jnp.float32),
            pltpu.VMEM((3, QPT), jnp.float32),
            pltpu.VMEM((GRP, N), jnp.int32),
            pltpu.VMEM((GRP, 256), jnp.int32),
            pltpu.VMEM((GRP, 16), jnp.int32),
            pltpu.VMEM((GRP, 64), jnp.int32),
            pltpu.VMEM((GRP, NB), jnp.int32),
            pltpu.VMEM((GRP, 128), jnp.float32),
            pltpu.VMEM((GRP, NB, FP), jnp.float32),
            pltpu.VMEM((GRP, N), jnp.int32),
            pltpu.VMEM((GRP, N), jnp.int32),
            pltpu.SemaphoreType.DMA,
        ],
    )(keys_t, points_t, feats_pad)


def _tc_body(rel4_ref, nf_ref, W1p_ref, b1t_ref, W2p_ref, b2t_ref,
             W3p_ref, b3t_ref, F1r_ref, fb1_ref, F2_ref, fb2_ref, out_ref,
             e_ref):
    rel4 = rel4_ref[...]
    h = jnp.maximum(jnp.dot(rel4, W1p_ref[...],
                            preferred_element_type=jnp.float32)
                    + b1t_ref[...][None, :], 0.0)
    h = jnp.maximum(jnp.dot(h, W2p_ref[...],
                            preferred_element_type=jnp.float32)
                    + b2t_ref[...][None, :], 0.0)
    m = jnp.dot(h, W3p_ref[...],
                preferred_element_type=jnp.float32) + b3t_ref[...][None, :]
    e_ref[...] = jnp.zeros((QB, C_MID, C_IN), jnp.float32)
    for j in range(NB):
        mj = m[:, j * C_MID:(j + 1) * C_MID]
        fj = nf_ref[:, j, 0:C_IN]
        e_ref[...] += mj[:, :, None] * fj[:, None, :]
    g = fb1_ref[...][None, :]
    for a in range(C_MID):
        g = g + jnp.dot(e_ref[:, a, :], F1r_ref[a],
                        preferred_element_type=jnp.float32)
    g = jnp.maximum(g, 0.0)
    out_ref[...] = jnp.dot(g, F2_ref[...],
                           preferred_element_type=jnp.float32) \
        + fb2_ref[...][None, :]


def _tc_mlp(rel4, nf3, W1p, b1t, W2p, b2t, W3p, b3t, F1r, fb1, F2, fb2):
    fixed = lambda *shape: pl.BlockSpec(shape, lambda i: (0,) * len(shape))
    return pl.pallas_call(
        _tc_body,
        grid=(NBLK,),
        in_specs=[
            pl.BlockSpec((QB, 128), lambda i: (i, 0)),
            pl.BlockSpec((QB, NB, FP), lambda i: (i, 0, 0)),
            fixed(128, 1024), fixed(1024), fixed(1024, 1024), fixed(1024),
            fixed(1024, 512), fixed(512), fixed(C_MID, C_IN, 256), fixed(256),
            fixed(256, C_OUT), fixed(C_OUT),
        ],
        out_specs=pl.BlockSpec((QB, C_OUT), lambda i: (i, 0)),
        out_shape=jax.ShapeDtypeStruct((BK, C_OUT), jnp.float32),
        scratch_shapes=[pltpu.VMEM((QB, C_MID, C_IN), jnp.float32)],
    )(rel4, nf3, W1p, b1t, W2p, b2t, W3p, b3t, F1r, fb1, F2, fb2)


def _prep_weights(W1, b1, W2, b2, W3, b3, F1):
    W1pad = jnp.concatenate([W1, jnp.zeros((1, 32), jnp.float32)], axis=0)
    W1p = jnp.kron(jnp.eye(NB, dtype=jnp.float32), W1pad)
    W2p = jnp.kron(jnp.eye(NB, dtype=jnp.float32), W2)
    W3p = jnp.kron(jnp.eye(NB, dtype=jnp.float32), W3)
    b1t = jnp.tile(b1, NB)
    b2t = jnp.tile(b2, NB)
    b3t = jnp.tile(b3, NB)
    F1r = F1.reshape(C_MID, C_IN, 256)
    return W1p, b1t, W2p, b2t, W3p, b3t, F1r


def kernel(keys, points, feats, W1, b1, W2, b2, W3, b3, F1, fb1, F2, fb2):
    keys_t = keys.transpose(0, 2, 1)
    points_t = points.transpose(0, 2, 1)
    feats_pad = jnp.pad(feats.reshape(B * N, C_IN),
                        ((0, 0), (0, FP - C_IN)))
    nf3, rel4 = _sc_select_gather(keys_t, points_t, feats_pad)
    W1p, b1t, W2p, b2t, W3p, b3t, F1r = _prep_weights(W1, b1, W2, b2, W3, b3,
                                                      F1)
    out = _tc_mlp(rel4, nf3, W1p, b1t, W2p, b2t, W3p, b3t, F1r, fb1, F2, fb2)
    return out.reshape(B, K, C_OUT)


# speedup vs baseline: 7.1629x; 1.1755x over previous
import functools

import jax
import jax.numpy as jnp
from jax import lax
from jax.experimental import pallas as pl
from jax.experimental.pallas import tpu as pltpu
from jax.experimental.pallas import tpu_sc as plsc

B, K, N, DIM = 4, 1024, 2048, 3
C_IN, C_MID, NB, C_OUT = 64, 16, 32, 128
BK = B * K

NW = 32
KH = K // 2
BKH = B * KH
QPT = BKH // NW
GRP = 8
NGRP = QPT // GRP
NV = N // 16
FP = 128

QB = 128
NBLK = BK // QB


def _sc_body(keys_hbm, pts_hbm, feats_hbm, nf_out, rel_out,
             pts_v, kq_v, bits8, hist8, chist8, idx8, gidx8, relb, fbuf,
             cb8, ci8, semg):
    iota = lax.iota(jnp.int32, 16)
    ones = jnp.ones((16,), jnp.int32)
    t32 = jnp.full((16,), NB, jnp.int32)
    zf16 = jnp.zeros((16,), jnp.float32)
    zi16 = jnp.zeros((16,), jnp.int32)

    wid = lax.axis_index("s") * 2 + lax.axis_index("c")
    b = wid // (KH // QPT)
    kbase = (wid % (KH // QPT)) * QPT
    feat_off = b * N

    pltpu.sync_copy(pts_hbm.at[b], pts_v)
    pltpu.sync_copy(keys_hbm.at[b], kq_v)

    gconsts = [jnp.full((16,), g, jnp.int32) for g in range(GRP)]

    def scan_radix(g, thresh):
        cvec = chist8[g, pl.ds(0, 16)]
        cumc = plsc.cumsum(cvec)
        cstar = plsc.all_reduce_population_count(cumc < thresh)
        belowc = jnp.take(cumc - cvec, cstar)
        hv = plsc.load_gather(hist8, [gconsts[g], cstar * 16 + iota])
        cumf = plsc.cumsum(hv) + belowc
        bloc = plsc.all_reduce_population_count(cumf < thresh)
        below = jnp.take(cumf - hv, bloc)
        return cstar * 16 + bloc, below

    def zero_hist(g):
        for t in range(16):
            hist8[g, pl.ds(t * 16, 16)] = zi16
        chist8[g, pl.ds(0, 16)] = zi16

    def grp_body(grp, carry):
        q0 = grp * GRP
        qs = []
        for g in range(GRP):
            qloc = jnp.full((16,), kbase + q0 + g, jnp.int32)
            qs.append([plsc.load_gather(kq_v, [jnp.full((16,), d, jnp.int32),
                                               qloc]) for d in range(3)])
        for g in range(GRP):
            zero_hist(g)

        def p1_body(i, c):
            sl = pl.ds(i * 16, 16)
            px = pts_v[0, sl]
            py = pts_v[1, sl]
            pz = pts_v[2, sl]
            for g in range(GRP):
                dx = px - qs[g][0]
                dy = py - qs[g][1]
                dz = pz - qs[g][2]
                d2 = dx * dx + dy * dy + dz * dz
                bt = plsc.bitcast(d2, jnp.int32)
                bits8[g, sl] = bt
                plsc.addupdate_scatter(
                    hist8, [gconsts[g], lax.shift_right_logical(bt, 24)],
                    ones)
                plsc.addupdate_scatter(
                    chist8, [gconsts[g], lax.shift_right_logical(bt, 28)],
                    ones)
            return c
        lax.fori_loop(0, NV, p1_body, 0)

        for g in range(GRP):
            b0, c1 = scan_radix(g, t32)
            t2 = t32 - c1
            zero_hist(g)

            def p2_body(i, carry):
                cur_lt, cur_c = carry
                bt = bits8[g, pl.ds(i * 16, 16)]
                top8 = lax.shift_right_logical(bt, 24)
                match = top8 == b0
                lt8 = top8 < b0
                bin2 = jnp.bitwise_and(lax.shift_right_logical(bt, 16), 255)
                plsc.addupdate_scatter(hist8, [gconsts[g], bin2], ones,
                                       mask=match)
                plsc.addupdate_scatter(
                    chist8, [gconsts[g], lax.shift_right_logical(bin2, 4)],
                    ones, mask=match)
                lanes = i * 16 + iota
                cs_lt = plsc.cumsum(lt8.astype(jnp.int32))
                plsc.store_scatter(idx8, [gconsts[g], cur_lt + cs_lt - 1],
                                   lanes, mask=lt8)
                cs_c = plsc.cumsum(match.astype(jnp.int32))
                posc = cur_c + cs_c - 1
                plsc.store_scatter(cb8, [gconsts[g], posc], bt, mask=match)
                plsc.store_scatter(ci8, [gconsts[g], posc], lanes,
                                   mask=match)
                return (cur_lt + plsc.all_reduce_population_count(lt8),
                        cur_c + plsc.all_reduce_population_count(match))
            cur_lt, cur_c = lax.fori_loop(0, NV, p2_body, (zi16, zi16),
                                          unroll=2)
            m_sc = cur_c[0]
            nvc = (m_sc + 15) // 16

            b1, c2 = scan_radix(g, t2)
            t3 = t2 - c2
            zero_hist(g)

            def p3_body(i, c):
                bt = cb8[g, pl.ds(i * 16, 16)]
                inb = (i * 16 + iota) < m_sc
                nxt8 = jnp.bitwise_and(lax.shift_right_logical(bt, 16), 255)
                match = jnp.logical_and(nxt8 == b1, inb)
                bin3 = jnp.bitwise_and(lax.shift_right_logical(bt, 8), 255)
                plsc.addupdate_scatter(hist8, [gconsts[g], bin3], ones,
                                       mask=match)
                plsc.addupdate_scatter(
                    chist8, [gconsts[g], lax.shift_right_logical(bin3, 4)],
                    ones, mask=match)
                return c
            lax.fori_loop(0, nvc, p3_body, 0)
            b2, c3 = scan_radix(g, t3)
            t4 = t3 - c3
            zero_hist(g)

            def p4_body(i, c):
                bt = cb8[g, pl.ds(i * 16, 16)]
                inb = (i * 16 + iota) < m_sc
                nxt8 = jnp.bitwise_and(lax.shift_right_logical(bt, 16), 255)
                mid8 = jnp.bitwise_and(lax.shift_right_logical(bt, 8), 255)
                match = jnp.logical_and(
                    jnp.logical_and(nxt8 == b1, mid8 == b2), inb)
                bin4 = jnp.bitwise_and(bt, 255)
                plsc.addupdate_scatter(hist8, [gconsts[g], bin4], ones,
                                       mask=match)
                plsc.addupdate_scatter(
                    chist8, [gconsts[g], lax.shift_right_logical(bin4, 4)],
                    ones, mask=match)
                return c
            lax.fori_loop(0, nvc, p4_body, 0)
            b3, _ = scan_radix(g, t4)
            T = ((b0 * 256 + b1) * 256 + b2) * 256 + b3

            def lt_body(i, cur):
                bt = cb8[g, pl.ds(i * 16, 16)]
                ci = ci8[g, pl.ds(i * 16, 16)]
                inb = (i * 16 + iota) < m_sc
                m = jnp.logical_and(bt < T, inb)
                cs = plsc.cumsum(m.astype(jnp.int32))
                pos = cur + cs - 1
                plsc.store_scatter(idx8, [gconsts[g], pos], ci, mask=m)
                return cur + plsc.all_reduce_population_count(m)
            cur = lax.fori_loop(0, nvc, lt_body, cur_lt)

            def eq_body(i, cur):
                bt = cb8[g, pl.ds(i * 16, 16)]
                ci = ci8[g, pl.ds(i * 16, 16)]
                inb = (i * 16 + iota) < m_sc
                m = jnp.logical_and(bt == T, inb)
                cs = plsc.cumsum(m.astype(jnp.int32))
                pos = cur + cs - 1
                keep = jnp.logical_and(m, pos < t32)
                plsc.store_scatter(idx8, [gconsts[g], pos], ci, mask=keep)
                return cur + plsc.all_reduce_population_count(m)
            lax.fori_loop(0, nvc, eq_body, cur)

            i0 = idx8[g, pl.ds(0, 16)]
            i1 = idx8[g, pl.ds(16, 16)]
            for d in range(3):
                dsp = jnp.full((16,), d, jnp.int32)
                v0 = plsc.load_gather(pts_v, [dsp, i0]) - qs[g][d]
                v1 = plsc.load_gather(pts_v, [dsp, i1]) - qs[g][d]
                plsc.store_scatter(relb, [gconsts[g], iota * 4 + d], v0)
                plsc.store_scatter(relb, [gconsts[g], (iota + 16) * 4 + d],
                                   v1)
            plsc.store_scatter(relb, [gconsts[g], iota * 4 + 3], zf16)
            plsc.store_scatter(relb, [gconsts[g], (iota + 16) * 4 + 3], zf16)
            gidx8[g, pl.ds(0, 16)] = i0 + feat_off
            gidx8[g, pl.ds(16, 16)] = i1 + feat_off

        descs = [pltpu.async_copy(feats_hbm.at[gidx8.at[g]], fbuf.at[g],
                                  semg) for g in range(GRP)]
        for dsc in descs:
            dsc.wait()
        row0 = wid * QPT + q0
        pltpu.sync_copy(fbuf, nf_out.at[pl.ds(row0, GRP)])
        pltpu.sync_copy(relb, rel_out.at[pl.ds(row0, GRP)])
        return carry

    lax.fori_loop(0, NGRP, grp_body, 0)


def _sc_select_gather(keys_t, points_t, feats_pad):
    mesh = plsc.VectorSubcoreMesh(core_axis_name="c", subcore_axis_name="s",
                                  num_cores=2, num_subcores=16)
    return pl.kernel(
        _sc_body,
        out_type=(jax.ShapeDtypeStruct((BKH, NB, FP), jnp.float32),
                  jax.ShapeDtypeStruct((BKH, 128), jnp.float32)),
        mesh=mesh,
        compiler_params=pltpu.CompilerParams(needs_layout_passes=False),
        scratch_types=[
            pltpu.VMEM((3, N), jnp.float32),
            pltpu.VMEM((3, KH), jnp.float32),
            pltpu.VMEM((GRP, N), jnp.int32),
            pltpu.VMEM((GRP, 256), jnp.int32),
            pltpu.VMEM((GRP, 16), jnp.int32),
            pltpu.VMEM((GRP, 64), jnp.int32),
            pltpu.VMEM((GRP, NB), jnp.int32),
            pltpu.VMEM((GRP, 128), jnp.float32),
            pltpu.VMEM((GRP, NB, FP), jnp.float32),
            pltpu.VMEM((GRP, N), jnp.int32),
            pltpu.VMEM((GRP, N), jnp.int32),
            pltpu.SemaphoreType.DMA,
        ],
    )(keys_t, points_t, feats_pad)


def _tc_body(rel4_ref, nf_ref, W1p_ref, b1t_ref, W2p_ref, b2t_ref,
             W3p_ref, b3t_ref, F1r_ref, fb1_ref, F2_ref, fb2_ref, out_ref,
             e_ref):
    rel4 = rel4_ref[...]
    h = jnp.maximum(jnp.dot(rel4, W1p_ref[...],
                            preferred_element_type=jnp.float32)
                    + b1t_ref[...][None, :], 0.0)
    h = jnp.maximum(jnp.dot(h, W2p_ref[...],
                            preferred_element_type=jnp.float32)
                    + b2t_ref[...][None, :], 0.0)
    m = jnp.dot(h, W3p_ref[...],
                preferred_element_type=jnp.float32) + b3t_ref[...][None, :]
    e_ref[...] = jnp.zeros((QB, C_MID, C_IN), jnp.float32)
    for j in range(NB):
        mj = m[:, j * C_MID:(j + 1) * C_MID]
        fj = nf_ref[:, j, 0:C_IN]
        e_ref[...] += mj[:, :, None] * fj[:, None, :]
    g = fb1_ref[...][None, :]
    for a in range(C_MID):
        g = g + jnp.dot(e_ref[:, a, :], F1r_ref[a],
                        preferred_element_type=jnp.float32)
    g = jnp.maximum(g, 0.0)
    out_ref[...] = jnp.dot(g, F2_ref[...],
                           preferred_element_type=jnp.float32) \
        + fb2_ref[...][None, :]


def _tc_mlp(rel4, nf3, W1p, b1t, W2p, b2t, W3p, b3t, F1r, fb1, F2, fb2):
    nrow = rel4.shape[0]
    fixed = lambda *shape: pl.BlockSpec(shape, lambda i: (0,) * len(shape))
    return pl.pallas_call(
        _tc_body,
        grid=(nrow // QB,),
        in_specs=[
            pl.BlockSpec((QB, 128), lambda i: (i, 0)),
            pl.BlockSpec((QB, NB, FP), lambda i: (i, 0, 0)),
            fixed(128, 1024), fixed(1024), fixed(1024, 1024), fixed(1024),
            fixed(1024, 512), fixed(512), fixed(C_MID, C_IN, 256), fixed(256),
            fixed(256, C_OUT), fixed(C_OUT),
        ],
        out_specs=pl.BlockSpec((QB, C_OUT), lambda i: (i, 0)),
        out_shape=jax.ShapeDtypeStruct((nrow, C_OUT), jnp.float32),
        scratch_shapes=[pltpu.VMEM((QB, C_MID, C_IN), jnp.float32)],
    )(rel4, nf3, W1p, b1t, W2p, b2t, W3p, b3t, F1r, fb1, F2, fb2)


def _prep_weights(W1, b1, W2, b2, W3, b3, F1):
    W1pad = jnp.concatenate([W1, jnp.zeros((1, 32), jnp.float32)], axis=0)
    W1p = jnp.kron(jnp.eye(NB, dtype=jnp.float32), W1pad)
    W2p = jnp.kron(jnp.eye(NB, dtype=jnp.float32), W2)
    W3p = jnp.kron(jnp.eye(NB, dtype=jnp.float32), W3)
    b1t = jnp.tile(b1, NB)
    b2t = jnp.tile(b2, NB)
    b3t = jnp.tile(b3, NB)
    F1r = F1.reshape(C_MID, C_IN, 256)
    return W1p, b1t, W2p, b2t, W3p, b3t, F1r


def kernel(keys, points, feats, W1, b1, W2, b2, W3, b3, F1, fb1, F2, fb2):
    keys_t = keys.transpose(0, 2, 1)
    points_t = points.transpose(0, 2, 1)
    feats_pad = jnp.pad(feats.reshape(B * N, C_IN),
                        ((0, 0), (0, FP - C_IN)))
    W1p, b1t, W2p, b2t, W3p, b3t, F1r = _prep_weights(W1, b1, W2, b2, W3, b3,
                                                      F1)
    outs = []
    for h in range(2):
        kh = keys_t[:, :, h * KH:(h + 1) * KH]
        nf3, rel4 = _sc_select_gather(kh, points_t, feats_pad)
        out_h = _tc_mlp(rel4, nf3, W1p, b1t, W2p, b2t, W3p, b3t, F1r,
                        fb1, F2, fb2)
        outs.append(out_h.reshape(B, KH, C_OUT))
    return jnp.concatenate(outs, axis=1)
